# Initial kernel scaffold; baseline (speedup 1.0000x reference)
import jax
import jax.numpy as jnp
from jax.experimental import pallas as pl

N_GRAPHS = 256
NPG = 196
N = N_GRAPHS * NPG
EPS = 1e-5
LOG_STD_MAX = 2.0
LOG_STD_MIN = -5.0

GB = 32
GRID = N_GRAPHS // GB

C1P = 8
C2P = 32
C3P = 36


def _mm(h3, w):
    c = h3.shape[-1]
    r = h3.reshape(-1, c) @ w
    return r.reshape(h3.shape[:-1] + (w.shape[1],))


def _graphnorm(h, w, b, a):
    mean = jnp.mean(h, axis=1, keepdims=True)
    out = h - mean * a
    var = jnp.mean(out * out, axis=1, keepdims=True)
    return w * out / jnp.sqrt(var + EPS) + b


def _full(shape):
    nd = len(shape)
    return pl.BlockSpec(shape, lambda i, _n=nd: (0,) * _n)


def _blk(c):
    return pl.BlockSpec((GB, NPG, c), lambda i: (i, 0, 0))


def _t0_body(cnt_ref, x_ref, dis_ref, g1_ref):
    deg = cnt_ref[...] + 1.0
    dis = jax.lax.rsqrt(deg)
    dis_ref[...] = dis
    x = x_ref[...]
    cols = jax.lax.broadcasted_iota(jnp.float32, (GB, NPG, C1P), 2)
    g1_ref[...] = jnp.where(cols == x, dis, 0.0) * (cols < 2.0)


def _t0(cnt, x3):
    return pl.pallas_call(
        _t0_body,
        grid=(GRID,),
        in_specs=[_blk(1), _blk(1)],
        out_specs=[_blk(1), _blk(C1P)],
        out_shape=[
            jax.ShapeDtypeStruct((N_GRAPHS, NPG, 1), jnp.float32),
            jax.ShapeDtypeStruct((N_GRAPHS, NPG, C1P), jnp.float32),
        ],
    )(cnt, x3)


def _t1_body(p1_ref, x_ref, dis_ref, e12_ref, w1_ref, b1_ref, gw_ref, gb_ref,
             ga_ref, h1_ref, g2_ref):
    dis = dis_ref[...]
    x = x_ref[...]
    e12 = e12_ref[...]
    h0 = jnp.where(x == 1.0, e12[1][None, None, :], e12[0][None, None, :])
    s2 = p1_ref[...][:, :, :2]
    agg = dis * _mm(s2, e12) + (dis * dis) * h0
    h = _mm(agg, w1_ref[...]) + b1_ref[...][0]
    h = _graphnorm(h, gw_ref[...][0], gb_ref[...][0], ga_ref[...][0])
    h1 = h + h0
    h1_ref[...] = h1
    g2 = dis * h1
    g2_ref[...] = jnp.concatenate(
        [g2, jnp.zeros((GB, NPG, C2P - 18), jnp.float32)], axis=-1)


def _t1(p1, x3, dis3, e12, w1, b1, gw, gb, ga):
    return pl.pallas_call(
        _t1_body,
        grid=(GRID,),
        in_specs=[_blk(C1P), _blk(1), _blk(1), _full((2, 18)), _full((18, 18)),
                  _full((1, 18)), _full((1, 18)), _full((1, 18)), _full((1, 18))],
        out_specs=[_blk(18), _blk(C2P)],
        out_shape=[
            jax.ShapeDtypeStruct((N_GRAPHS, NPG, 18), jnp.float32),
            jax.ShapeDtypeStruct((N_GRAPHS, NPG, C2P), jnp.float32),
        ],
    )(p1, x3, dis3, e12, w1, b1, gw, gb, ga)


def _t2_body(p2_ref, h1_ref, dis_ref, w2_ref, b2_ref, gw_ref, gb_ref, ga_ref,
             rw_ref, rb_ref, h2_ref, g3_ref):
    dis = dis_ref[...]
    h1 = h1_ref[...]
    agg = dis * p2_ref[...][:, :, :18] + (dis * dis) * h1
    h = _mm(agg, w2_ref[...]) + b2_ref[...][0]
    h = _graphnorm(h, gw_ref[...][0], gb_ref[...][0], ga_ref[...][0])
    res1 = _mm(h1, rw_ref[...]) + rb_ref[...][0]
    h2 = h + res1
    h2_ref[...] = h2
    g3_ref[...] = dis * h2


def _t2(p2, h1, dis3, w2, b2, gw, gb, ga, rw, rb):
    return pl.pallas_call(
        _t2_body,
        grid=(GRID,),
        in_specs=[_blk(C2P), _blk(18), _blk(1), _full((18, 36)), _full((1, 36)),
                  _full((1, 36)), _full((1, 36)), _full((1, 36)),
                  _full((18, 36)), _full((1, 36))],
        out_specs=[_blk(36), _blk(C3P)],
        out_shape=[
            jax.ShapeDtypeStruct((N_GRAPHS, NPG, 36), jnp.float32),
            jax.ShapeDtypeStruct((N_GRAPHS, NPG, C3P), jnp.float32),
        ],
    )(p2, h1, dis3, w2, b2, gw, gb, ga, rw, rb)


def _t3_body(p3_ref, h2_ref, dis_ref, w3_ref, b3_ref, gw_ref, gb_ref, ga_ref,
             rw_ref, rb_ref, f1w_ref, f1b_ref, f2w_ref, f2b_ref, fmw_ref,
             fmb_ref, flw_ref, flb_ref, mean_ref, lstd_ref):
    dis = dis_ref[...]
    h2 = h2_ref[...]
    agg = dis * p3_ref[...] + (dis * dis) * h2
    h = _mm(agg, w3_ref[...]) + b3_ref[...][0]
    h = _graphnorm(h, gw_ref[...][0], gb_ref[...][0], ga_ref[...][0])
    res2 = _mm(h2, rw_ref[...]) + rb_ref[...][0]
    h3 = h + res2
    pooled = jnp.max(h3, axis=1)
    z = jax.nn.relu(pooled @ f1w_ref[...] + f1b_ref[...][0])
    z = jax.nn.relu(z @ f2w_ref[...] + f2b_ref[...][0])
    mean_ref[...] = z @ fmw_ref[...] + fmb_ref[...][0]
    lstd = jnp.tanh(z @ flw_ref[...] + flb_ref[...][0])
    lstd_ref[...] = LOG_STD_MIN + 0.5 * (LOG_STD_MAX - LOG_STD_MIN) * (lstd + 1.0)


def _t3(p3, h2, dis3, w3, b3, gw, gb, ga, rw, rb, f1w, f1b, f2w, f2b, fmw,
        fmb, flw, flb):
    ospec = pl.BlockSpec((GB, 1), lambda i: (i, 0))
    return pl.pallas_call(
        _t3_body,
        grid=(GRID,),
        in_specs=[_blk(C3P), _blk(36), _blk(1), _full((36, 72)), _full((1, 72)),
                  _full((1, 72)), _full((1, 72)), _full((1, 72)),
                  _full((36, 72)), _full((1, 72)),
                  _full((72, 256)), _full((1, 256)), _full((256, 256)),
                  _full((1, 256)), _full((256, 1)), _full((1, 1)),
                  _full((256, 1)), _full((1, 1))],
        out_specs=[ospec, ospec],
        out_shape=[
            jax.ShapeDtypeStruct((N_GRAPHS, 1), jnp.float32),
            jax.ShapeDtypeStruct((N_GRAPHS, 1), jnp.float32),
        ],
    )(p3, h2, dis3, w3, b3, gw, gb, ga, rw, rb, f1w, f1b, f2w, f2b, fmw,
      fmb, flw, flb)


def _edge_counts(dst):
    return jax.ops.segment_sum(jnp.ones((dst.shape[0],), jnp.float32), dst,
                               num_segments=N)


def _propagate(g, src, dst):
    return jax.ops.segment_sum(g[src], dst, num_segments=N)


def kernel(x, edge_index, batch, emb, conv1_w, conv1_b, conv2_w, conv2_b,
           conv3_w, conv3_b, res1_w, res1_b, res2_w, res2_b, gn1_w, gn1_b,
           gn1_a, gn2_w, gn2_b, gn2_a, gn3_w, gn3_b, gn3_a, fc1_w, fc1_b,
           fc2_w, fc2_b, fcm_w, fcm_b, fcl_w, fcl_b):
    del batch
    src = edge_index[0].astype(jnp.int32)
    dst = edge_index[1].astype(jnp.int32)
    x3 = x.astype(jnp.float32).reshape(N_GRAPHS, NPG, 1)
    r2 = lambda v: v.reshape(1, -1)

    cnt = _edge_counts(dst).reshape(N_GRAPHS, NPG, 1)
    dis3, g1 = _t0(cnt, x3)

    p1 = _propagate(g1.reshape(N, C1P), src, dst).reshape(N_GRAPHS, NPG, C1P)
    h1, g2 = _t1(p1, x3, dis3, emb[1:3], conv1_w, r2(conv1_b), r2(gn1_w),
                 r2(gn1_b), r2(gn1_a))

    p2 = _propagate(g2.reshape(N, C2P), src, dst).reshape(N_GRAPHS, NPG, C2P)
    h2, g3 = _t2(p2, h1, dis3, conv2_w, r2(conv2_b), r2(gn2_w), r2(gn2_b),
                 r2(gn2_a), res1_w, r2(res1_b))

    p3 = _propagate(g3.reshape(N, C3P), src, dst).reshape(N_GRAPHS, NPG, C3P)
    mean_out, log_std = _t3(p3, h2, dis3, conv3_w, r2(conv3_b), r2(gn3_w),
                            r2(gn3_b), r2(gn3_a), res2_w, r2(res2_b), fc1_w,
                            r2(fc1_b), fc2_w, r2(fc2_b), fcm_w, r2(fcm_b),
                            fcl_w, r2(fcl_b))
    return (mean_out, log_std)


# speedup vs baseline: 2.9034x; 2.9034x over previous
import jax
import jax.numpy as jnp
from jax.experimental import pallas as pl

N_GRAPHS = 256
NPG = 196
N = N_GRAPHS * NPG
EPS = 1e-5
LOG_STD_MAX = 2.0
LOG_STD_MIN = -5.0

GB = 32
GRID = N_GRAPHS // GB

C1P = 8
C2P = 32
C3P = 36


_PREC = jax.lax.Precision.HIGHEST


def _mm2(a, w, prec=None):
    return jnp.dot(a, w, precision=prec, preferred_element_type=jnp.float32)


def _mm(h3, w, prec=None):
    c = h3.shape[-1]
    r = _mm2(h3.reshape(-1, c), w, prec)
    return r.reshape(h3.shape[:-1] + (w.shape[1],))


def _graphnorm(h, w, b, a):
    mean = jnp.mean(h, axis=1, keepdims=True)
    out = h - mean * a
    var = jnp.mean(out * out, axis=1, keepdims=True)
    return w * out / jnp.sqrt(var + EPS) + b


def _full(shape):
    nd = len(shape)
    return pl.BlockSpec(shape, lambda i, _n=nd: (0,) * _n)


def _blk(c):
    return pl.BlockSpec((GB, NPG, c), lambda i: (i, 0, 0))


def _t0_body(cnt_ref, x_ref, dis_ref, g1_ref):
    deg = cnt_ref[...] + 1.0
    dis = jax.lax.rsqrt(deg)
    dis_ref[...] = dis
    x = x_ref[...]
    cols = jax.lax.broadcasted_iota(jnp.int32, (GB, NPG, C1P), 2).astype(
        jnp.float32)
    g1_ref[...] = jnp.where((cols == x) & (cols < 2.0), dis, 0.0)


def _t0(cnt, x3):
    return pl.pallas_call(
        _t0_body,
        grid=(GRID,),
        in_specs=[_blk(1), _blk(1)],
        out_specs=[_blk(1), _blk(C1P)],
        out_shape=[
            jax.ShapeDtypeStruct((N_GRAPHS, NPG, 1), jnp.float32),
            jax.ShapeDtypeStruct((N_GRAPHS, NPG, C1P), jnp.float32),
        ],
    )(cnt, x3)


def _t1_body(p1_ref, x_ref, dis_ref, e12_ref, w1_ref, b1_ref, gw_ref, gb_ref,
             ga_ref, h1_ref, g2_ref):
    dis = dis_ref[...]
    x = x_ref[...]
    e12 = e12_ref[...]
    h0 = jnp.where(x == 1.0, e12[1][None, None, :], e12[0][None, None, :])
    s2 = p1_ref[...][:, :, :2]
    agg = dis * _mm(s2, e12, _PREC) + (dis * dis) * h0
    h = _mm(agg, w1_ref[...]) + b1_ref[...][0]
    h = _graphnorm(h, gw_ref[...][0], gb_ref[...][0], ga_ref[...][0])
    h1 = h + h0
    h1_ref[...] = h1
    g2 = dis * h1
    g2_ref[...] = jnp.concatenate(
        [g2, jnp.zeros((GB, NPG, C2P - 18), jnp.float32)], axis=-1)


def _t1(p1, x3, dis3, e12, w1, b1, gw, gb, ga):
    return pl.pallas_call(
        _t1_body,
        grid=(GRID,),
        in_specs=[_blk(C1P), _blk(1), _blk(1), _full((2, 18)), _full((18, 18)),
                  _full((1, 18)), _full((1, 18)), _full((1, 18)), _full((1, 18))],
        out_specs=[_blk(18), _blk(C2P)],
        out_shape=[
            jax.ShapeDtypeStruct((N_GRAPHS, NPG, 18), jnp.float32),
            jax.ShapeDtypeStruct((N_GRAPHS, NPG, C2P), jnp.float32),
        ],
    )(p1, x3, dis3, e12, w1, b1, gw, gb, ga)


def _t2_body(p2_ref, h1_ref, dis_ref, w2_ref, b2_ref, gw_ref, gb_ref, ga_ref,
             rw_ref, rb_ref, h2_ref, g3_ref):
    dis = dis_ref[...]
    h1 = h1_ref[...]
    agg = dis * p2_ref[...][:, :, :18] + (dis * dis) * h1
    h = _mm(agg, w2_ref[...]) + b2_ref[...][0]
    h = _graphnorm(h, gw_ref[...][0], gb_ref[...][0], ga_ref[...][0])
    res1 = _mm(h1, rw_ref[...]) + rb_ref[...][0]
    h2 = h + res1
    h2_ref[...] = h2
    g3_ref[...] = dis * h2


def _t2(p2, h1, dis3, w2, b2, gw, gb, ga, rw, rb):
    return pl.pallas_call(
        _t2_body,
        grid=(GRID,),
        in_specs=[_blk(C2P), _blk(18), _blk(1), _full((18, 36)), _full((1, 36)),
                  _full((1, 36)), _full((1, 36)), _full((1, 36)),
                  _full((18, 36)), _full((1, 36))],
        out_specs=[_blk(36), _blk(C3P)],
        out_shape=[
            jax.ShapeDtypeStruct((N_GRAPHS, NPG, 36), jnp.float32),
            jax.ShapeDtypeStruct((N_GRAPHS, NPG, C3P), jnp.float32),
        ],
    )(p2, h1, dis3, w2, b2, gw, gb, ga, rw, rb)


def _t3_body(p3_ref, h2_ref, dis_ref, w3_ref, b3_ref, gw_ref, gb_ref, ga_ref,
             rw_ref, rb_ref, f1w_ref, f1b_ref, f2w_ref, f2b_ref, fmw_ref,
             fmb_ref, flw_ref, flb_ref, mean_ref, lstd_ref):
    dis = dis_ref[...]
    h2 = h2_ref[...]
    agg = dis * p3_ref[...] + (dis * dis) * h2
    h = _mm(agg, w3_ref[...]) + b3_ref[...][0]
    h = _graphnorm(h, gw_ref[...][0], gb_ref[...][0], ga_ref[...][0])
    res2 = _mm(h2, rw_ref[...]) + rb_ref[...][0]
    h3 = h + res2
    pooled = jnp.max(h3, axis=1)
    z = jax.nn.relu(_mm2(pooled, f1w_ref[...]) + f1b_ref[...][0])
    z = jax.nn.relu(_mm2(z, f2w_ref[...]) + f2b_ref[...][0])
    mean_ref[...] = _mm2(z, fmw_ref[...]) + fmb_ref[...][0]
    lstd = jnp.tanh(_mm2(z, flw_ref[...]) + flb_ref[...][0])
    lstd_ref[...] = LOG_STD_MIN + 0.5 * (LOG_STD_MAX - LOG_STD_MIN) * (lstd + 1.0)


def _t3(p3, h2, dis3, w3, b3, gw, gb, ga, rw, rb, f1w, f1b, f2w, f2b, fmw,
        fmb, flw, flb):
    ospec = pl.BlockSpec((GB, 1), lambda i: (i, 0))
    return pl.pallas_call(
        _t3_body,
        grid=(GRID,),
        in_specs=[_blk(C3P), _blk(36), _blk(1), _full((36, 72)), _full((1, 72)),
                  _full((1, 72)), _full((1, 72)), _full((1, 72)),
                  _full((36, 72)), _full((1, 72)),
                  _full((72, 256)), _full((1, 256)), _full((256, 256)),
                  _full((1, 256)), _full((256, 1)), _full((1, 1)),
                  _full((256, 1)), _full((1, 1))],
        out_specs=[ospec, ospec],
        out_shape=[
            jax.ShapeDtypeStruct((N_GRAPHS, 1), jnp.float32),
            jax.ShapeDtypeStruct((N_GRAPHS, 1), jnp.float32),
        ],
    )(p3, h2, dis3, w3, b3, gw, gb, ga, rw, rb, f1w, f1b, f2w, f2b, fmw,
      fmb, flw, flb)


def _edge_counts(dst):
    return jax.ops.segment_sum(jnp.ones((dst.shape[0],), jnp.float32), dst,
                               num_segments=N)


def _propagate(g, src, dst):
    return jax.ops.segment_sum(g[src], dst, num_segments=N)


def kernel(x, edge_index, batch, emb, conv1_w, conv1_b, conv2_w, conv2_b,
           conv3_w, conv3_b, res1_w, res1_b, res2_w, res2_b, gn1_w, gn1_b,
           gn1_a, gn2_w, gn2_b, gn2_a, gn3_w, gn3_b, gn3_a, fc1_w, fc1_b,
           fc2_w, fc2_b, fcm_w, fcm_b, fcl_w, fcl_b):
    del batch
    src = edge_index[0].astype(jnp.int32)
    dst = edge_index[1].astype(jnp.int32)
    x3 = x.astype(jnp.float32).reshape(N_GRAPHS, NPG, 1)
    r2 = lambda v: v.reshape(1, -1)

    cnt = _edge_counts(dst).reshape(N_GRAPHS, NPG, 1)
    dis3, g1 = _t0(cnt, x3)

    p1 = _propagate(g1.reshape(N, C1P), src, dst).reshape(N_GRAPHS, NPG, C1P)
    h1, g2 = _t1(p1, x3, dis3, emb[1:3], conv1_w, r2(conv1_b), r2(gn1_w),
                 r2(gn1_b), r2(gn1_a))

    p2 = _propagate(g2.reshape(N, C2P), src, dst).reshape(N_GRAPHS, NPG, C2P)
    h2, g3 = _t2(p2, h1, dis3, conv2_w, r2(conv2_b), r2(gn2_w), r2(gn2_b),
                 r2(gn2_a), res1_w, r2(res1_b))

    p3 = _propagate(g3.reshape(N, C3P), src, dst).reshape(N_GRAPHS, NPG, C3P)
    mean_out, log_std = _t3(p3, h2, dis3, conv3_w, r2(conv3_b), r2(gn3_w),
                            r2(gn3_b), r2(gn3_a), res2_w, r2(res2_b), fc1_w,
                            r2(fc1_b), fc2_w, r2(fc2_b), fcm_w, r2(fcm_b),
                            fcl_w, r2(fcl_b))
    return (mean_out, log_std)


# speedup vs baseline: 22.3551x; 7.6995x over previous
import functools

import jax
import jax.numpy as jnp
from jax import lax
from jax.experimental import pallas as pl
from jax.experimental.pallas import tpu as pltpu
from jax.experimental.pallas import tpu_sc as plsc

N_GRAPHS = 256
NPG = 196
N = N_GRAPHS * NPG
EPS = 1e-5
LOG_STD_MAX = 2.0
LOG_STD_MIN = -5.0

GB = 32
GRID = N_GRAPHS // GB

C1P = 8
C2P = 32
C3A = 24
C3B = 16


_PREC = jax.lax.Precision.HIGHEST


def _mm2(a, w, prec=None):
    return jnp.dot(a, w, precision=prec, preferred_element_type=jnp.float32)


def _mm(h3, w, prec=None):
    c = h3.shape[-1]
    r = _mm2(h3.reshape(-1, c), w, prec)
    return r.reshape(h3.shape[:-1] + (w.shape[1],))


def _graphnorm(h, w, b, a):
    mean = jnp.mean(h, axis=1, keepdims=True)
    out = h - mean * a
    var = jnp.mean(out * out, axis=1, keepdims=True)
    return w * out / jnp.sqrt(var + EPS) + b


def _full(shape):
    nd = len(shape)
    return pl.BlockSpec(shape, lambda i, _n=nd: (0,) * _n)


def _blk(c):
    return pl.BlockSpec((GB, NPG, c), lambda i: (i, 0, 0))


def _t0_body(cnt_ref, x_ref, dis_ref, g1_ref):
    deg = cnt_ref[...][:, :, :1] + 1.0
    dis = jax.lax.rsqrt(deg)
    dis_ref[...] = dis
    x = x_ref[...]
    cols = jax.lax.broadcasted_iota(jnp.int32, (GB, NPG, C1P), 2).astype(
        jnp.float32)
    g1_ref[...] = jnp.where((cols == x) & (cols < 2.0), dis, 0.0)


def _t0(cnt, x3):
    return pl.pallas_call(
        _t0_body,
        grid=(GRID,),
        in_specs=[_blk(8), _blk(1)],
        out_specs=[_blk(1), _blk(C1P)],
        out_shape=[
            jax.ShapeDtypeStruct((N_GRAPHS, NPG, 1), jnp.float32),
            jax.ShapeDtypeStruct((N_GRAPHS, NPG, C1P), jnp.float32),
        ],
    )(cnt, x3)


def _t1_body(p1_ref, x_ref, dis_ref, e12_ref, w1_ref, b1_ref, gw_ref, gb_ref,
             ga_ref, h1_ref, g2_ref):
    dis = dis_ref[...]
    x = x_ref[...]
    e12 = e12_ref[...]
    h0 = jnp.where(x == 1.0, e12[1][None, None, :], e12[0][None, None, :])
    s2 = p1_ref[...][:, :, :2]
    agg = dis * _mm(s2, e12, _PREC) + (dis * dis) * h0
    h = _mm(agg, w1_ref[...]) + b1_ref[...][0]
    h = _graphnorm(h, gw_ref[...][0], gb_ref[...][0], ga_ref[...][0])
    h1 = h + h0
    h1_ref[...] = h1
    g2 = dis * h1
    g2_ref[...] = jnp.concatenate(
        [g2, jnp.zeros((GB, NPG, C2P - 18), jnp.float32)], axis=-1)


def _t1(p1, x3, dis3, e12, w1, b1, gw, gb, ga):
    return pl.pallas_call(
        _t1_body,
        grid=(GRID,),
        in_specs=[_blk(C1P), _blk(1), _blk(1), _full((2, 18)), _full((18, 18)),
                  _full((1, 18)), _full((1, 18)), _full((1, 18)), _full((1, 18))],
        out_specs=[_blk(18), _blk(C2P)],
        out_shape=[
            jax.ShapeDtypeStruct((N_GRAPHS, NPG, 18), jnp.float32),
            jax.ShapeDtypeStruct((N_GRAPHS, NPG, C2P), jnp.float32),
        ],
    )(p1, x3, dis3, e12, w1, b1, gw, gb, ga)


def _t2_body(p2_ref, h1_ref, dis_ref, w2_ref, b2_ref, gw_ref, gb_ref, ga_ref,
             rw_ref, rb_ref, h2_ref, g3a_ref, g3b_ref):
    dis = dis_ref[...]
    h1 = h1_ref[...]
    agg = dis * p2_ref[...][:, :, :18] + (dis * dis) * h1
    h = _mm(agg, w2_ref[...]) + b2_ref[...][0]
    h = _graphnorm(h, gw_ref[...][0], gb_ref[...][0], ga_ref[...][0])
    res1 = _mm(h1, rw_ref[...]) + rb_ref[...][0]
    h2 = h + res1
    h2_ref[...] = h2
    g3 = dis * h2
    g3a_ref[...] = g3[:, :, :C3A]
    g3b_ref[...] = jnp.concatenate(
        [g3[:, :, C3A:], jnp.zeros((GB, NPG, C3B - 12), jnp.float32)],
        axis=-1)


def _t2(p2, h1, dis3, w2, b2, gw, gb, ga, rw, rb):
    return pl.pallas_call(
        _t2_body,
        grid=(GRID,),
        in_specs=[_blk(C2P), _blk(18), _blk(1), _full((18, 36)), _full((1, 36)),
                  _full((1, 36)), _full((1, 36)), _full((1, 36)),
                  _full((18, 36)), _full((1, 36))],
        out_specs=[_blk(36), _blk(C3A), _blk(C3B)],
        out_shape=[
            jax.ShapeDtypeStruct((N_GRAPHS, NPG, 36), jnp.float32),
            jax.ShapeDtypeStruct((N_GRAPHS, NPG, C3A), jnp.float32),
            jax.ShapeDtypeStruct((N_GRAPHS, NPG, C3B), jnp.float32),
        ],
    )(p2, h1, dis3, w2, b2, gw, gb, ga, rw, rb)


def _t3_body(p3a_ref, p3b_ref, h2_ref, dis_ref, w3_ref, b3_ref, gw_ref, gb_ref, ga_ref,
             rw_ref, rb_ref, f1w_ref, f1b_ref, f2w_ref, f2b_ref, fmw_ref,
             fmb_ref, flw_ref, flb_ref, mean_ref, lstd_ref):
    dis = dis_ref[...]
    h2 = h2_ref[...]
    p3 = jnp.concatenate([p3a_ref[...], p3b_ref[...][:, :, :12]], axis=-1)
    agg = dis * p3 + (dis * dis) * h2
    h = _mm(agg, w3_ref[...]) + b3_ref[...][0]
    h = _graphnorm(h, gw_ref[...][0], gb_ref[...][0], ga_ref[...][0])
    res2 = _mm(h2, rw_ref[...]) + rb_ref[...][0]
    h3 = h + res2
    pooled = jnp.max(h3, axis=1)
    z = jax.nn.relu(_mm2(pooled, f1w_ref[...]) + f1b_ref[...][0])
    z = jax.nn.relu(_mm2(z, f2w_ref[...]) + f2b_ref[...][0])
    mean_ref[...] = _mm2(z, fmw_ref[...]) + fmb_ref[...][0]
    lstd = jnp.tanh(_mm2(z, flw_ref[...]) + flb_ref[...][0])
    lstd_ref[...] = LOG_STD_MIN + 0.5 * (LOG_STD_MAX - LOG_STD_MIN) * (lstd + 1.0)


def _t3(p3a, p3b, h2, dis3, w3, b3, gw, gb, ga, rw, rb, f1w, f1b, f2w, f2b,
        fmw, fmb, flw, flb):
    ospec = pl.BlockSpec((GB, 1), lambda i: (i, 0))
    return pl.pallas_call(
        _t3_body,
        grid=(GRID,),
        in_specs=[_blk(C3A), _blk(C3B), _blk(36), _blk(1),
                  _full((36, 72)), _full((1, 72)),
                  _full((1, 72)), _full((1, 72)), _full((1, 72)),
                  _full((36, 72)), _full((1, 72)),
                  _full((72, 256)), _full((1, 256)), _full((256, 256)),
                  _full((1, 256)), _full((256, 1)), _full((1, 1)),
                  _full((256, 1)), _full((1, 1))],
        out_specs=[ospec, ospec],
        out_shape=[
            jax.ShapeDtypeStruct((N_GRAPHS, 1), jnp.float32),
            jax.ShapeDtypeStruct((N_GRAPHS, 1), jnp.float32),
        ],
    )(p3a, p3b, h2, dis3, w3, b3, gw, gb, ga, rw, rb, f1w, f1b, f2w, f2b,
      fmw, fmb, flw, flb)


E = 401408
NTILES = 32
EPT = E // NTILES
CHUNK = 128
NCH = EPT // CHUNK
NBLK = 7
BCH = NCH // NBLK
RPT = N // 16


def _sc_mesh():
    return plsc.VectorSubcoreMesh(core_axis_name="c", subcore_axis_name="s")


_SC_PARAMS = pltpu.CompilerParams(use_tc_tiling_on_sc=False)


def _sc_deg(dst_flat, ones_h, zrows):
    @functools.partial(
        pl.kernel,
        out_type=jax.ShapeDtypeStruct((2, N, 8), jnp.float32),
        mesh=_sc_mesh(),
        compiler_params=_SC_PARAMS,
        scratch_types=[
            pltpu.VMEM((CHUNK,), jnp.int32),
            pltpu.VMEM((CHUNK,), jnp.int32),
            pltpu.VMEM((CHUNK, 8), jnp.float32),
            pltpu.SemaphoreType.DMA,
            pltpu.SemaphoreType.DMA,
            pltpu.VMEM_SHARED((N, 8), jnp.float32),
        ],
    )
    def k(dst_hbm, ones_hbm, z_hbm, out_hbm, dstc0, dstc1, ones_v,
          semd0, semd1, acc_sh):
        c = lax.axis_index("c")
        s = lax.axis_index("s")
        wid = s * 2 + c
        base = wid * EPT
        pltpu.sync_copy(z_hbm, acc_sh.at[pl.ds(s * RPT, RPT)])
        pltpu.sync_copy(ones_hbm, ones_v)
        plsc.subcore_barrier()
        pltpu.async_copy(dst_hbm.at[pl.ds(base, CHUNK)], dstc0, semd0)
        pltpu.async_copy(dst_hbm.at[pl.ds(base + CHUNK, CHUNK)], dstc1, semd1)

        @pl.loop(0, NCH, step=2)
        def _(j):
            off = base + j * CHUNK
            pltpu.make_async_copy(dst_hbm.at[pl.ds(off, CHUNK)], dstc0,
                                  semd0).wait()
            pltpu.sync_copy(ones_v, acc_sh.at[dstc0], add=True)

            @pl.when(j + 2 < NCH)
            def _():
                pltpu.async_copy(dst_hbm.at[pl.ds(off + 2 * CHUNK, CHUNK)],
                                 dstc0, semd0)

            pltpu.make_async_copy(dst_hbm.at[pl.ds(off + CHUNK, CHUNK)],
                                  dstc1, semd1).wait()
            pltpu.sync_copy(ones_v, acc_sh.at[dstc1], add=True)

            @pl.when(j + 3 < NCH)
            def _():
                pltpu.async_copy(dst_hbm.at[pl.ds(off + 3 * CHUNK, CHUNK)],
                                 dstc1, semd1)

        plsc.subcore_barrier()
        pltpu.sync_copy(acc_sh.at[pl.ds(s * RPT, RPT)],
                        out_hbm.at[c].at[pl.ds(s * RPT, RPT)])

    return k(dst_flat, ones_h, zrows)


def _sc_scatter(g, src4, dst_flat, zrows, cp):
    @functools.partial(
        pl.kernel,
        out_type=jax.ShapeDtypeStruct((2, N, cp), jnp.float32),
        mesh=_sc_mesh(),
        compiler_params=_SC_PARAMS,
        scratch_types=[
            pltpu.VMEM((BCH, CHUNK), jnp.int32),
            pltpu.VMEM((CHUNK,), jnp.int32),
            pltpu.VMEM((CHUNK,), jnp.int32),
            pltpu.VMEM((CHUNK, cp), jnp.float32),
            pltpu.VMEM((CHUNK, cp), jnp.float32),
            pltpu.SemaphoreType.DMA,
            pltpu.SemaphoreType.DMA,
            pltpu.SemaphoreType.DMA,
            pltpu.SemaphoreType.DMA,
            pltpu.VMEM_SHARED((N, cp), jnp.float32),
        ],
    )
    def k(g_hbm, src_hbm, dst_hbm, z_hbm, out_hbm,
          src_v, dstc0, dstc1, rows0, rows1, sem0, sem1, semd0, semd1,
          acc_sh):
        c = lax.axis_index("c")
        s = lax.axis_index("s")
        wid = s * 2 + c
        pltpu.sync_copy(z_hbm, acc_sh.at[pl.ds(s * RPT, RPT)])
        plsc.subcore_barrier()
        for blk in range(NBLK):
            base = wid * EPT + blk * (BCH * CHUNK)
            pltpu.sync_copy(src_hbm.at[wid].at[blk], src_v)
            pltpu.async_copy(g_hbm.at[src_v.at[0]], rows0, sem0)
            pltpu.async_copy(g_hbm.at[src_v.at[1]], rows1, sem1)
            pltpu.async_copy(dst_hbm.at[pl.ds(base, CHUNK)], dstc0, semd0)
            pltpu.async_copy(dst_hbm.at[pl.ds(base + CHUNK, CHUNK)], dstc1,
                             semd1)

            @pl.loop(0, BCH, step=2)
            def _(j):
                off = base + j * CHUNK
                pltpu.make_async_copy(g_hbm.at[src_v.at[j]], rows0,
                                      sem0).wait()
                pltpu.make_async_copy(dst_hbm.at[pl.ds(off, CHUNK)], dstc0,
                                      semd0).wait()
                pltpu.sync_copy(rows0, acc_sh.at[dstc0], add=True)

                @pl.when(j + 2 < BCH)
                def _():
                    pltpu.async_copy(g_hbm.at[src_v.at[j + 2]], rows0, sem0)
                    pltpu.async_copy(
                        dst_hbm.at[pl.ds(off + 2 * CHUNK, CHUNK)], dstc0,
                        semd0)

                pltpu.make_async_copy(g_hbm.at[src_v.at[j + 1]], rows1,
                                      sem1).wait()
                pltpu.make_async_copy(dst_hbm.at[pl.ds(off + CHUNK, CHUNK)],
                                      dstc1, semd1).wait()
                pltpu.sync_copy(rows1, acc_sh.at[dstc1], add=True)

                @pl.when(j + 3 < BCH)
                def _():
                    pltpu.async_copy(g_hbm.at[src_v.at[j + 3]], rows1, sem1)
                    pltpu.async_copy(
                        dst_hbm.at[pl.ds(off + 3 * CHUNK, CHUNK)], dstc1,
                        semd1)

        plsc.subcore_barrier()
        pltpu.sync_copy(acc_sh.at[pl.ds(s * RPT, RPT)],
                        out_hbm.at[c].at[pl.ds(s * RPT, RPT)])

    return k(g, src4, dst_flat, zrows)


def _edge_counts(dst_flat):
    ones_h = jnp.ones((CHUNK, 8), jnp.float32)
    zrows = jnp.zeros((RPT, 8), jnp.float32)
    acc = _sc_deg(dst_flat, ones_h, zrows)
    return acc[0] + acc[1]


def _propagate(g, src4, dst_flat):
    cp = g.shape[1]
    zrows = jnp.zeros((RPT, cp), jnp.float32)
    acc = _sc_scatter(g, src4, dst_flat, zrows, cp)
    return acc[0] + acc[1]


def kernel(x, edge_index, batch, emb, conv1_w, conv1_b, conv2_w, conv2_b,
           conv3_w, conv3_b, res1_w, res1_b, res2_w, res2_b, gn1_w, gn1_b,
           gn1_a, gn2_w, gn2_b, gn2_a, gn3_w, gn3_b, gn3_a, fc1_w, fc1_b,
           fc2_w, fc2_b, fcm_w, fcm_b, fcl_w, fcl_b):
    del batch
    src = edge_index[0].astype(jnp.int32)
    dst = edge_index[1].astype(jnp.int32)
    src4 = src.reshape(NTILES, NBLK, BCH, CHUNK)

    x3 = x.astype(jnp.float32).reshape(N_GRAPHS, NPG, 1)
    r2 = lambda v: v.reshape(1, -1)

    cnt = _edge_counts(dst).reshape(N_GRAPHS, NPG, 8)
    dis3, g1 = _t0(cnt, x3)

    p1 = _propagate(g1.reshape(N, C1P), src4, dst).reshape(N_GRAPHS, NPG, C1P)
    h1, g2 = _t1(p1, x3, dis3, emb[1:3], conv1_w, r2(conv1_b), r2(gn1_w),
                 r2(gn1_b), r2(gn1_a))

    p2 = _propagate(g2.reshape(N, C2P), src4, dst).reshape(N_GRAPHS, NPG, C2P)
    h2, g3a, g3b = _t2(p2, h1, dis3, conv2_w, r2(conv2_b), r2(gn2_w),
                       r2(gn2_b), r2(gn2_a), res1_w, r2(res1_b))

    p3a = _propagate(g3a.reshape(N, C3A), src4, dst).reshape(N_GRAPHS, NPG, C3A)
    p3b = _propagate(g3b.reshape(N, C3B), src4, dst).reshape(N_GRAPHS, NPG, C3B)
    mean_out, log_std = _t3(p3a, p3b, h2, dis3, conv3_w, r2(conv3_b),
                            r2(gn3_w), r2(gn3_b), r2(gn3_a), res2_w,
                            r2(res2_b), fc1_w, r2(fc1_b), fc2_w, r2(fc2_b),
                            fcm_w, r2(fcm_b), fcl_w, r2(fcl_b))
    return (mean_out, log_std)
